# Initial kernel scaffold; baseline (speedup 1.0000x reference)
import jax
import jax.numpy as jnp
from jax.experimental import pallas as pl


def kernel(x, W_r, w_c_fc, w_gate, w_c_proj):
    raise NotImplementedError("write your pallas kernel here")


# speedup vs baseline: 1.7783x; 1.7783x over previous
import functools

import jax
import jax.numpy as jnp
from jax.experimental import pallas as pl
from jax.experimental.pallas import tpu as pltpu

N_EXPERTS = 8
TOP_K = 2
LOAD_FACTOR = 1.25


def _route_body(x_ref, wr_ref, prob_ref, idx_ref, pos_ref, *, B, T, cap):
    x = x_ref[...]
    logits = jnp.dot(x, wr_ref[...], preferred_element_type=jnp.float32)
    m = jnp.max(logits, axis=-1, keepdims=True)
    ex = jnp.exp(logits - m)
    probs = ex / jnp.sum(ex, axis=-1, keepdims=True)

    lane = jax.lax.broadcasted_iota(jnp.int32, probs.shape, 1)
    m0 = jnp.max(probs, axis=-1, keepdims=True)
    e0 = jnp.min(jnp.where(probs == m0, lane, N_EXPERTS), axis=-1, keepdims=True)
    pm = jnp.where(lane == e0, -jnp.inf, probs)
    m1 = jnp.max(pm, axis=-1, keepdims=True)
    e1 = jnp.min(jnp.where(pm == m1, lane, N_EXPERTS), axis=-1, keepdims=True)

    oh0 = (lane == e0).astype(jnp.float32)
    oh1 = (lane == e1).astype(jnp.float32)

    r = jax.lax.broadcasted_iota(jnp.int32, (T, T), 0)
    c = jax.lax.broadcasted_iota(jnp.int32, (T, T), 1)
    stril = (r > c).astype(jnp.float32)

    for b in range(B):
        sl = slice(b * T, (b + 1) * T)
        o0 = oh0[sl]
        o1 = oh1[sl]
        c0 = jnp.dot(stril, o0, preferred_element_type=jnp.float32)
        c1 = jnp.dot(stril, o1, preferred_element_type=jnp.float32)
        tot0 = jnp.sum(o0, axis=0, keepdims=True)
        pos0 = jnp.sum(c0 * o0, axis=-1)
        pos1 = jnp.sum((c1 + tot0) * o1, axis=-1)
        pos_ref[b, 0, :] = pos0.astype(jnp.int32)
        pos_ref[b, 1, :] = pos1.astype(jnp.int32)
        idx_ref[b, 0, :] = e0[sl, 0]
        idx_ref[b, 1, :] = e1[sl, 0]
        prob_ref[b, 0, :] = m0[sl, 0]
        prob_ref[b, 1, :] = m1[sl, 0]


def _moe_body(x_ref, w1_ref, w2_ref, w3_ref, prob_ref, idx_ref, pos_ref,
              y_ref, xe_ref, oacc_ref, *, B, T, C, cap, n_h):
    e = pl.program_id(0)
    h = pl.program_id(1)

    @pl.when(h == 0)
    def _dispatch():
        piota = jax.lax.broadcasted_iota(jnp.int32, (cap, T), 0)
        for b in range(B):
            msum = None
            for k in range(TOP_K):
                idxk = idx_ref[b, k, :][None, :]
                posk = pos_ref[b, k, :][None, :]
                mk = ((idxk == e) & (posk == piota)).astype(jnp.float32)
                msum = mk if msum is None else msum + mk
            xe_ref[b * cap:(b + 1) * cap, :] = jnp.dot(
                msum, x_ref[b * T:(b + 1) * T, :],
                preferred_element_type=jnp.float32)

    xe = xe_ref[...]
    w1 = w1_ref[0]
    w2 = w2_ref[0]
    hh = jnp.dot(xe, w1, preferred_element_type=jnp.float32)
    gg = jnp.dot(xe, w2, preferred_element_type=jnp.float32)
    gg = gg * (1.0 / (1.0 + jnp.exp(-gg)))
    act = gg * hh
    o_tile = jnp.dot(act, w3_ref[0], preferred_element_type=jnp.float32)

    @pl.when(h == 0)
    def _init_o():
        oacc_ref[...] = o_tile

    @pl.when(h != 0)
    def _acc_o():
        oacc_ref[...] = oacc_ref[...] + o_tile

    @pl.when(h == n_h - 1)
    def _combine():
        ciota = jax.lax.broadcasted_iota(jnp.int32, (T, cap), 1)
        for b in range(B):
            csum = None
            for k in range(TOP_K):
                idxk = idx_ref[b, k, :][:, None]
                posk = jnp.minimum(pos_ref[b, k, :], cap - 1)[:, None]
                pk = prob_ref[b, k, :][:, None]
                ck = jnp.where((idxk == e) & (posk == ciota), pk, 0.0)
                csum = ck if csum is None else csum + ck
            contrib = jnp.dot(csum, oacc_ref[b * cap:(b + 1) * cap, :],
                              preferred_element_type=jnp.float32)

            @pl.when(e == 0)
            def _init_y():
                y_ref[b * T:(b + 1) * T, :] = contrib

            @pl.when(e != 0)
            def _acc_y():
                y_ref[b * T:(b + 1) * T, :] = y_ref[b * T:(b + 1) * T, :] + contrib


def kernel(x, W_r, w_c_fc, w_gate, w_c_proj):
    B, T, C = x.shape
    E, _, H = w_c_fc.shape
    cap = int(LOAD_FACTOR * TOP_K * max(1, T / E))
    xf = x.reshape(B * T, C)

    prob, idx, pos = pl.pallas_call(
        functools.partial(_route_body, B=B, T=T, cap=cap),
        out_shape=(
            jax.ShapeDtypeStruct((B, TOP_K, T), jnp.float32),
            jax.ShapeDtypeStruct((B, TOP_K, T), jnp.int32),
            jax.ShapeDtypeStruct((B, TOP_K, T), jnp.int32),
        ),
    )(xf, W_r)

    HT = 512
    n_h = H // HT
    grid = (E, n_h)
    y = pl.pallas_call(
        functools.partial(_moe_body, B=B, T=T, C=C, cap=cap, n_h=n_h),
        grid=grid,
        in_specs=[
            pl.BlockSpec((B * T, C), lambda e, h: (0, 0)),
            pl.BlockSpec((1, C, HT), lambda e, h: (e, 0, h)),
            pl.BlockSpec((1, C, HT), lambda e, h: (e, 0, h)),
            pl.BlockSpec((1, HT, C), lambda e, h: (e, h, 0)),
            pl.BlockSpec((B, TOP_K, T), lambda e, h: (0, 0, 0)),
            pl.BlockSpec((B, TOP_K, T), lambda e, h: (0, 0, 0)),
            pl.BlockSpec((B, TOP_K, T), lambda e, h: (0, 0, 0)),
        ],
        out_specs=pl.BlockSpec((B * T, C), lambda e, h: (0, 0)),
        out_shape=jax.ShapeDtypeStruct((B * T, C), jnp.float32),
        scratch_shapes=[
            pltpu.VMEM((B * cap, C), jnp.float32),
            pltpu.VMEM((B * cap, C), jnp.float32),
        ],
        compiler_params=pltpu.CompilerParams(
            vmem_limit_bytes=100 * 1024 * 1024,
        ),
    )(xf, w_c_fc, w_gate, w_c_proj, prob, idx, pos)

    return y.reshape(B, T, C)
